# baseline (speedup 1.0000x reference)
import functools
import math

import jax
import jax.numpy as jnp
import numpy as np
from jax import lax
from jax.experimental import pallas as pl
from jax.experimental.pallas import tpu as pltpu
from jax.experimental.pallas import tpu_sc as plsc

N = 10000
E = 320000
D = 128
NRBF = 20
NI = 3
CUTOFF = 5.0
LN2 = math.log(2.0)

NC = 2
NS = 16
NW = NC * NS
L = 16

NP = 10240
NPW = NP // NW
NPL = N // NS
EPW = E // NW
C = 80
KC = EPW // C
SLAB = 25
NSLAB = KC // SLAB
PAIRS = (SLAB - 1) // 2
GC = NPW // C

NRBF_P = 32
BE = 2560
BN = 640

_sc_mesh = plsc.VectorSubcoreMesh(
    core_axis_name="c", subcore_axis_name="s", num_cores=NC, num_subcores=NS)
_sc_params = pltpu.CompilerParams(
    needs_layout_passes=False, use_tc_tiling_on_sc=False)


@functools.partial(
    pl.kernel,
    out_type=(
        jax.ShapeDtypeStruct((NP, D), jnp.float32),
        jax.ShapeDtypeStruct((E,), jnp.float32),
    ),
    mesh=_sc_mesh,
    compiler_params=_sc_params,
    scratch_types=[
        pltpu.VMEM((NPW,), jnp.int32),
        pltpu.VMEM((NPW, D), jnp.float32),
        pltpu.VMEM((N * 3,), jnp.float32),
        pltpu.VMEM((EPW,), jnp.int32),
        pltpu.VMEM((EPW,), jnp.int32),
        pltpu.VMEM((EPW,), jnp.float32),
        pltpu.SemaphoreType.DMA,
    ],
)
def _sc_embed_geom(an_hbm, emb_hbm, r_hbm, ii_hbm, jj_hbm,
                   x0_hbm, d2_hbm,
                   an_v, xrows_v, r_v, ii_v, jj_v, d2_v, sem):
    cid = lax.axis_index("c")
    sid = lax.axis_index("s")
    wid = sid * NC + cid

    nbase = wid * NPW
    pltpu.sync_copy(an_hbm.at[pl.ds(nbase, NPW)], an_v)
    for g in range(GC):
        pltpu.async_copy(
            emb_hbm.at[an_v.at[pl.ds(g * C, C)]],
            xrows_v.at[pl.ds(g * C, C)], sem).wait()
    pltpu.sync_copy(xrows_v, x0_hbm.at[pl.ds(nbase, NPW)])

    ebase = wid * EPW
    pltpu.sync_copy(r_hbm, r_v)
    pltpu.sync_copy(ii_hbm.at[pl.ds(ebase, EPW)], ii_v)
    pltpu.sync_copy(jj_hbm.at[pl.ds(ebase, EPW)], jj_v)
    def body(e, carry):
        sl = pl.ds(e * L, L)
        iv = ii_v[sl] * 3
        jv = jj_v[sl] * 3
        xi = plsc.load_gather(r_v, [iv])
        yi = plsc.load_gather(r_v, [iv + 1])
        zi = plsc.load_gather(r_v, [iv + 2])
        xj = plsc.load_gather(r_v, [jv])
        yj = plsc.load_gather(r_v, [jv + 1])
        zj = plsc.load_gather(r_v, [jv + 2])
        dx = xj - xi
        dy = yj - yi
        dz = zj - zi
        d2_v[sl] = dx * dx + dy * dy + dz * dz
        return carry

    lax.fori_loop(0, EPW // L, body, None)
    pltpu.sync_copy(d2_v, d2_hbm.at[pl.ds(ebase, EPW)])


def _tc_rcut_body(d2_ref, rc_ref):
    d = jnp.sqrt(d2_ref[...] + 1e-12)
    rc = 0.5 * (jnp.cos(d * (math.pi / CUTOFF)) + 1.0)
    rc_ref[...] = rc * (d < CUTOFF).astype(jnp.float32)


_tc_rcut = pl.pallas_call(
    _tc_rcut_body,
    out_shape=jax.ShapeDtypeStruct((E // 128, 128), jnp.float32),
)


def _tc_filter_body(d2_ref, rc_ref, cen_ref, coe_ref, w1_ref, b1_ref, w2_ref,
                    b2_ref, out_ref):
    d = jnp.sqrt(d2_ref[...] + 1e-12)
    diff = d - cen_ref[...]
    f = jnp.exp(coe_ref[...] * diff * diff)
    g = jnp.dot(f, w1_ref[...], preferred_element_type=jnp.float32) + b1_ref[...]
    a = jax.nn.softplus(g) - LN2
    w = jnp.dot(a, w2_ref[...], preferred_element_type=jnp.float32) + b2_ref[...]
    out_ref[...] = (w * rc_ref[...]).astype(jnp.bfloat16)


_tc_filter = pl.pallas_call(
    _tc_filter_body,
    grid=(E // BE,),
    in_specs=[
        pl.BlockSpec((BE, 1), lambda i: (i, 0)),
        pl.BlockSpec((BE, 1), lambda i: (i, 0)),
        pl.BlockSpec((1, NRBF_P), lambda i: (0, 0)),
        pl.BlockSpec((1, NRBF_P), lambda i: (0, 0)),
        pl.BlockSpec((NRBF_P, D), lambda i: (0, 0)),
        pl.BlockSpec((1, D), lambda i: (0, 0)),
        pl.BlockSpec((D, D), lambda i: (0, 0)),
        pl.BlockSpec((1, D), lambda i: (0, 0)),
    ],
    out_specs=pl.BlockSpec((BE, D), lambda i: (i, 0)),
    out_shape=jax.ShapeDtypeStruct((E, D), jnp.bfloat16),
)


@functools.partial(
    pl.kernel,
    out_type=jax.ShapeDtypeStruct((NC, NP, D), jnp.float32),
    mesh=_sc_mesh,
    compiler_params=_sc_params,
    scratch_types=[
        pltpu.VMEM((SLAB, C), jnp.int32),
        pltpu.VMEM((SLAB, C), jnp.int32),
        pltpu.VMEM((SLAB, C), jnp.int32),
        pltpu.VMEM((SLAB, C), jnp.int32),
        pltpu.VMEM((C, D), jnp.float32),
        pltpu.VMEM((C, D), jnp.float32),
        pltpu.VMEM((C, D), jnp.bfloat16),
        pltpu.VMEM_SHARED((N, D), jnp.float32),
        pltpu.SemaphoreType.DMA,
        pltpu.SemaphoreType.DMA,
        pltpu.SemaphoreType.DMA,
        pltpu.SemaphoreType.DMA,
    ],
)
def _sc_conv(h_hbm, wt_hbm, ii_hbm, jj_hbm, out_hbm,
             ii0, jj0, ii1, jj1, hbuf0, hbuf1, wbuf, agg, g0, g1, s0, s1):
    cid = lax.axis_index("c")
    sid = lax.axis_index("s")
    wid = sid * NC + cid
    ebase = wid * EPW

    def zrow(r, carry):
        for c8 in range(D // L):
            hbuf0[r, pl.ds(c8 * L, L)] = jnp.zeros((L,), jnp.float32)
            hbuf1[r, pl.ds(c8 * L, L)] = jnp.zeros((L,), jnp.float32)
        return carry

    lax.fori_loop(0, C, zrow, None)
    for z in range(NPL // C):
        pltpu.sync_copy(hbuf0, agg.at[pl.ds(sid * NPL + z * C, C)])
    pltpu.sync_copy(hbuf0.at[pl.ds(0, NPL % C)],
                    agg.at[pl.ds(sid * NPL + (NPL // C) * C, NPL % C)])
    pltpu.sync_copy(ii_hbm.at[wid].at[pl.ds(0, SLAB)], ii0)
    pltpu.sync_copy(jj_hbm.at[wid].at[pl.ds(0, SLAB)], jj0)
    plsc.subcore_barrier()

    pltpu.async_copy(hbuf0, agg.at[ii0.at[0]], s0, add=True)
    pltpu.async_copy(hbuf1, agg.at[ii0.at[0]], s1, add=True)
    pltpu.make_async_copy(hbuf0, agg.at[ii0.at[0]], s0).wait()
    pltpu.async_copy(h_hbm.at[jj0.at[0]], hbuf0, g0)

    bufs = (hbuf0, hbuf1)
    gsems = (g0, g1)
    ssems = (s0, s1)
    slabs = ((ii0, jj0), (ii1, jj1))

    def mul(buf):
        def mrow(r2, c2):
            for dr in range(2):
                r = r2 * 2 + dr
                for c in range(D // (2 * L)):
                    wv = wbuf[r, pl.ds(c * 2 * L, 2 * L)]
                    wa, wb = plsc.unpack(wv, format=plsc.PackFormat.INTERLEAVED)
                    sl0 = pl.ds(c * 2 * L, L)
                    sl1 = pl.ds(c * 2 * L + L, L)
                    buf[r, sl0] = buf[r, sl0] * wa
                    buf[r, sl1] = buf[r, sl1] * wb
            return c2

        lax.fori_loop(0, C // 2, mrow, None)

    for sb in range(NSLAB):
        sa = sb % 2
        iiP, jjP = slabs[sa]
        iiQ, jjQ = slabs[1 - sa]
        bA, bB = bufs[sa], bufs[1 - sa]
        gA, gB = gsems[sa], gsems[1 - sa]
        sA, sB = ssems[sa], ssems[1 - sa]

        def chunk_op(lk, buf, gsem, ssem, iiX, jjX, prefetch):
            off = pl.multiple_of(ebase + (sb * SLAB) * C + lk * C, 8)
            pltpu.sync_copy(wt_hbm.at[pl.ds(off, C)], wbuf)
            pltpu.make_async_copy(h_hbm.at[jjX.at[lk]], buf, gsem).wait()
            prefetch()
            mul(buf)
            pltpu.async_copy(buf, agg.at[iiX.at[lk]], ssem, add=True)

        def pair(k2, carry):
            a = 2 * k2

            def pref_a():
                pltpu.make_async_copy(bB, agg.at[iiP.at[a]], sB).wait()
                pltpu.async_copy(h_hbm.at[jjP.at[a + 1]], bB, gB)

            chunk_op(a, bA, gA, sA, iiP, jjP, pref_a)

            def pref_b():
                pltpu.make_async_copy(bA, agg.at[iiP.at[a + 1]], sA).wait()
                pltpu.async_copy(h_hbm.at[jjP.at[a + 2]], bA, gA)

            chunk_op(a + 1, bB, gB, sB, iiP, jjP, pref_b)
            return carry

        lax.fori_loop(0, PAIRS, pair, None)

        lk_tail = SLAB - 1
        if sb < NSLAB - 1:
            def pref_tail():
                pltpu.sync_copy(ii_hbm.at[wid].at[pl.ds((sb + 1) * SLAB, SLAB)], iiQ)
                pltpu.sync_copy(jj_hbm.at[wid].at[pl.ds((sb + 1) * SLAB, SLAB)], jjQ)
                pltpu.make_async_copy(bB, agg.at[iiP.at[lk_tail]], sB).wait()
                pltpu.async_copy(h_hbm.at[jjQ.at[0]], bB, gB)
        else:
            def pref_tail():
                pass

        chunk_op(lk_tail, bA, gA, sA, iiP, jjP, pref_tail)

    pltpu.make_async_copy(hbuf0, agg.at[ii0.at[0]], s0).wait()
    pltpu.make_async_copy(hbuf1, agg.at[ii0.at[0]], s1).wait()
    plsc.subcore_barrier()

    osl = pl.ds(sid * NPL, NPL)
    pltpu.sync_copy(agg.at[osl], out_hbm.at[cid].at[osl])


def _tc_update_body(p_ref, w1_ref, b1_ref, w2_ref, b2_ref, x_ref, wn_ref,
                    xo_ref, ho_ref):
    aggb = p_ref[0] + p_ref[1]
    g = jnp.dot(aggb, w1_ref[...], preferred_element_type=jnp.float32) + b1_ref[...]
    v = jnp.dot(jax.nn.softplus(g) - LN2, w2_ref[...],
                preferred_element_type=jnp.float32) + b2_ref[...]
    xn = x_ref[...] + v
    xo_ref[...] = xn
    ho_ref[...] = jnp.dot(xn, wn_ref[...], preferred_element_type=jnp.float32)


_tc_update = pl.pallas_call(
    _tc_update_body,
    grid=(NP // BN,),
    in_specs=[
        pl.BlockSpec((NC, BN, D), lambda i: (0, i, 0)),
        pl.BlockSpec((D, D), lambda i: (0, 0)),
        pl.BlockSpec((1, D), lambda i: (0, 0)),
        pl.BlockSpec((D, D), lambda i: (0, 0)),
        pl.BlockSpec((1, D), lambda i: (0, 0)),
        pl.BlockSpec((BN, D), lambda i: (i, 0)),
        pl.BlockSpec((D, D), lambda i: (0, 0)),
    ],
    out_specs=[
        pl.BlockSpec((BN, D), lambda i: (i, 0)),
        pl.BlockSpec((BN, D), lambda i: (i, 0)),
    ],
    out_shape=[
        jax.ShapeDtypeStruct((NP, D), jnp.float32),
        jax.ShapeDtypeStruct((NP, D), jnp.float32),
    ],
)


def _tc_matmul_body(x_ref, w_ref, o_ref):
    o_ref[...] = jnp.dot(x_ref[...], w_ref[...],
                         preferred_element_type=jnp.float32)


_tc_matmul = pl.pallas_call(
    _tc_matmul_body,
    grid=(NP // BN,),
    in_specs=[
        pl.BlockSpec((BN, D), lambda i: (i, 0)),
        pl.BlockSpec((D, D), lambda i: (0, 0)),
    ],
    out_specs=pl.BlockSpec((BN, D), lambda i: (i, 0)),
    out_shape=jax.ShapeDtypeStruct((NP, D), jnp.float32),
)


def kernel(atomic_numbers, R, idx_i, idx_j, offsets, emb, in2f_W, f2out_W1,
           f2out_b1, f2out_W2, f2out_b2, filt_W1, filt_b1, filt_W2, filt_b2):
    del offsets
    an = atomic_numbers.astype(jnp.int32)
    an_pad = jnp.concatenate([an, jnp.zeros((NP - N,), jnp.int32)])
    ii = idx_i.astype(jnp.int32)
    jj = idx_j.astype(jnp.int32)

    x0, d2 = _sc_embed_geom(an_pad, emb, R.reshape(N * 3), ii, jj)
    d2c = d2.reshape(E, 1)
    rcc = _tc_rcut(d2.reshape(E // 128, 128)).reshape(E, 1)

    centers = jnp.concatenate(
        [jnp.linspace(0.0, CUTOFF, NRBF),
         jnp.zeros((NRBF_P - NRBF,))]).reshape(1, NRBF_P).astype(jnp.float32)
    width = CUTOFF / (NRBF - 1)
    coeff = jnp.concatenate(
        [jnp.full((NRBF,), -0.5 / width**2),
         jnp.zeros((NRBF_P - NRBF,))]).reshape(1, NRBF_P).astype(jnp.float32)
    w1p = jnp.concatenate(
        [filt_W1, jnp.zeros((NI, NRBF_P - NRBF, D), jnp.float32)], axis=1)

    qperm = np.empty((D,), np.int32)
    for c in range(D // 32):
        for i in range(16):
            qperm[32 * c + 2 * i] = 32 * c + i
            qperm[32 * c + 2 * i + 1] = 32 * c + 16 + i
    w2q = filt_W2[:, :, qperm]
    b2q = filt_b2[:, qperm]

    ii3 = ii.reshape(NW, KC, C)
    jj3 = jj.reshape(NW, KC, C)

    wts = [
        _tc_filter(d2c, rcc, centers, coeff, w1p[t],
                   filt_b1[t].reshape(1, D), w2q[t],
                   b2q[t].reshape(1, D))
        for t in range(NI)
    ]
    x = x0
    h = _tc_matmul(x, in2f_W[0])
    for t in range(NI):
        wt = wts[t]
        parts = _sc_conv(h, wt, ii3, jj3)
        x, h = _tc_update(parts, f2out_W1[t], f2out_b1[t].reshape(1, D),
                          f2out_W2[t], f2out_b2[t].reshape(1, D),
                          x, in2f_W[(t + 1) % NI])
    return x[:N]


# speedup vs baseline: 1.6674x; 1.6674x over previous
import functools
import math

import jax
import jax.numpy as jnp
import numpy as np
from jax import lax
from jax.experimental import pallas as pl
from jax.experimental.pallas import tpu as pltpu
from jax.experimental.pallas import tpu_sc as plsc

N = 10000
E = 320000
D = 128
NRBF = 20
NI = 3
CUTOFF = 5.0
LN2 = math.log(2.0)

NC = 2
NS = 16
NW = NC * NS
L = 16

NP = 10240
NPW = NP // NW
NPL = N // NS
EPW = E // NW
C = 80
KC = EPW // C
SLAB = 25
NSLAB = KC // SLAB
PAIRS = (SLAB - 1) // 2
GC = NPW // C

NRBF_P = 32
BE = 2560
BN = 640

_sc_mesh = plsc.VectorSubcoreMesh(
    core_axis_name="c", subcore_axis_name="s", num_cores=NC, num_subcores=NS)
_sc_params = pltpu.CompilerParams(
    needs_layout_passes=False, use_tc_tiling_on_sc=False)


@functools.partial(
    pl.kernel,
    out_type=(
        jax.ShapeDtypeStruct((NP, D), jnp.float32),
        jax.ShapeDtypeStruct((E,), jnp.float32),
    ),
    mesh=_sc_mesh,
    compiler_params=_sc_params,
    scratch_types=[
        pltpu.VMEM((NPW,), jnp.int32),
        pltpu.VMEM((NPW, D), jnp.float32),
        pltpu.VMEM((N * 3,), jnp.float32),
        pltpu.VMEM((EPW,), jnp.int32),
        pltpu.VMEM((EPW,), jnp.int32),
        pltpu.VMEM((EPW,), jnp.float32),
        pltpu.SemaphoreType.DMA,
    ],
)
def _sc_embed_geom(an_hbm, emb_hbm, r_hbm, ii_hbm, jj_hbm,
                   x0_hbm, d2_hbm,
                   an_v, xrows_v, r_v, ii_v, jj_v, d2_v, sem):
    cid = lax.axis_index("c")
    sid = lax.axis_index("s")
    wid = sid * NC + cid

    nbase = wid * NPW
    pltpu.sync_copy(an_hbm.at[pl.ds(nbase, NPW)], an_v)
    for g in range(GC):
        pltpu.async_copy(
            emb_hbm.at[an_v.at[pl.ds(g * C, C)]],
            xrows_v.at[pl.ds(g * C, C)], sem).wait()
    pltpu.sync_copy(xrows_v, x0_hbm.at[pl.ds(nbase, NPW)])

    ebase = wid * EPW
    pltpu.sync_copy(r_hbm, r_v)
    pltpu.sync_copy(ii_hbm.at[pl.ds(ebase, EPW)], ii_v)
    pltpu.sync_copy(jj_hbm.at[pl.ds(ebase, EPW)], jj_v)
    def body(e, carry):
        sl = pl.ds(e * L, L)
        iv = ii_v[sl] * 3
        jv = jj_v[sl] * 3
        xi = plsc.load_gather(r_v, [iv])
        yi = plsc.load_gather(r_v, [iv + 1])
        zi = plsc.load_gather(r_v, [iv + 2])
        xj = plsc.load_gather(r_v, [jv])
        yj = plsc.load_gather(r_v, [jv + 1])
        zj = plsc.load_gather(r_v, [jv + 2])
        dx = xj - xi
        dy = yj - yi
        dz = zj - zi
        d2_v[sl] = dx * dx + dy * dy + dz * dz
        return carry

    lax.fori_loop(0, EPW // L, body, None)
    pltpu.sync_copy(d2_v, d2_hbm.at[pl.ds(ebase, EPW)])


def _tc_rcut_body(d2_ref, rc_ref):
    d = jnp.sqrt(d2_ref[...] + 1e-12)
    rc = 0.5 * (jnp.cos(d * (math.pi / CUTOFF)) + 1.0)
    rc_ref[...] = rc * (d < CUTOFF).astype(jnp.float32)


_tc_rcut = pl.pallas_call(
    _tc_rcut_body,
    out_shape=jax.ShapeDtypeStruct((E // 128, 128), jnp.float32),
)


def _tc_filter_body(d2_ref, rc_ref, cen_ref, coe_ref, w1_ref, b1_ref, w2_ref,
                    b2_ref, out_ref):
    d = jnp.sqrt(d2_ref[...] + 1e-12)
    diff = d - cen_ref[...]
    f = jnp.exp(coe_ref[...] * diff * diff)
    g = jnp.dot(f, w1_ref[...], preferred_element_type=jnp.float32) + b1_ref[...]
    a = jax.nn.softplus(g) - LN2
    w = jnp.dot(a, w2_ref[...], preferred_element_type=jnp.float32) + b2_ref[...]
    out_ref[...] = w * rc_ref[...]


_tc_filter = pl.pallas_call(
    _tc_filter_body,
    grid=(E // BE,),
    in_specs=[
        pl.BlockSpec((BE, 1), lambda i: (i, 0)),
        pl.BlockSpec((BE, 1), lambda i: (i, 0)),
        pl.BlockSpec((1, NRBF_P), lambda i: (0, 0)),
        pl.BlockSpec((1, NRBF_P), lambda i: (0, 0)),
        pl.BlockSpec((NRBF_P, D), lambda i: (0, 0)),
        pl.BlockSpec((1, D), lambda i: (0, 0)),
        pl.BlockSpec((D, D), lambda i: (0, 0)),
        pl.BlockSpec((1, D), lambda i: (0, 0)),
    ],
    out_specs=pl.BlockSpec((BE, D), lambda i: (i, 0)),
    out_shape=jax.ShapeDtypeStruct((E, D), jnp.float32),
)


@functools.partial(
    pl.kernel,
    out_type=jax.ShapeDtypeStruct((NC, NP, D), jnp.float32),
    mesh=_sc_mesh,
    compiler_params=_sc_params,
    scratch_types=[
        pltpu.VMEM((SLAB, C), jnp.int32),
        pltpu.VMEM((SLAB, C), jnp.int32),
        pltpu.VMEM((SLAB, C), jnp.int32),
        pltpu.VMEM((SLAB, C), jnp.int32),
        pltpu.VMEM((C, D), jnp.float32),
        pltpu.VMEM((C, D), jnp.float32),
        pltpu.VMEM((C // 2, D), jnp.float32),
        pltpu.VMEM((C // 2, D), jnp.float32),
        pltpu.VMEM_SHARED((N, D), jnp.float32),
        pltpu.SemaphoreType.DMA,
        pltpu.SemaphoreType.DMA,
        pltpu.SemaphoreType.DMA,
        pltpu.SemaphoreType.DMA,
        pltpu.SemaphoreType.DMA,
        pltpu.SemaphoreType.DMA,
    ],
)
def _sc_conv(h_hbm, wt_hbm, ii_hbm, jj_hbm, out_hbm,
             ii0, jj0, ii1, jj1, hbuf0, hbuf1, wb0, wb1, agg,
             g0, g1, s0, s1, w0, w1):
    cid = lax.axis_index("c")
    sid = lax.axis_index("s")
    wid = sid * NC + cid
    ebase = wid * EPW

    def zrow(r, carry):
        for c8 in range(D // L):
            hbuf0[r, pl.ds(c8 * L, L)] = jnp.zeros((L,), jnp.float32)
            hbuf1[r, pl.ds(c8 * L, L)] = jnp.zeros((L,), jnp.float32)
        return carry

    lax.fori_loop(0, C, zrow, None)
    for z in range(NPL // C):
        pltpu.sync_copy(hbuf0, agg.at[pl.ds(sid * NPL + z * C, C)])
    pltpu.sync_copy(hbuf0.at[pl.ds(0, NPL % C)],
                    agg.at[pl.ds(sid * NPL + (NPL // C) * C, NPL % C)])
    pltpu.sync_copy(ii_hbm.at[wid].at[pl.ds(0, SLAB)], ii0)
    pltpu.sync_copy(jj_hbm.at[wid].at[pl.ds(0, SLAB)], jj0)
    plsc.subcore_barrier()

    pltpu.async_copy(hbuf0, agg.at[ii0.at[0]], s0, add=True)
    pltpu.async_copy(hbuf1, agg.at[ii0.at[0]], s1, add=True)
    pltpu.make_async_copy(hbuf0, agg.at[ii0.at[0]], s0).wait()
    pltpu.async_copy(h_hbm.at[jj0.at[0]], hbuf0, g0)

    HC = C // 2
    wbs = (wb0, wb1)
    wsems = (w0, w1)

    def wt_issue(abs_chunk, half):
        off = pl.multiple_of(ebase + abs_chunk * C + half * HC, 8)
        pltpu.async_copy(wt_hbm.at[pl.ds(off, HC)], wbs[half], wsems[half])

    def wt_wait(half):
        off0 = pl.multiple_of(ebase, 8)
        pltpu.make_async_copy(
            wt_hbm.at[pl.ds(off0, HC)], wbs[half], wsems[half]).wait()

    wt_issue(0, 0)
    wt_issue(0, 1)

    bufs = (hbuf0, hbuf1)
    gsems = (g0, g1)
    ssems = (s0, s1)
    slabs = ((ii0, jj0), (ii1, jj1))

    def mul_half(buf, wb, base):
        def mrow(r2, c2):
            for dr in range(2):
                r = r2 * 2 + dr
                for c8 in range(D // L):
                    sl = pl.ds(c8 * L, L)
                    buf[base + r, sl] = buf[base + r, sl] * wb[r, sl]
            return c2

        lax.fori_loop(0, HC // 2, mrow, None)

    for sb in range(NSLAB):
        sa = sb % 2
        iiP, jjP = slabs[sa]
        iiQ, jjQ = slabs[1 - sa]
        bA, bB = bufs[sa], bufs[1 - sa]
        gA, gB = gsems[sa], gsems[1 - sa]
        sA, sB = ssems[sa], ssems[1 - sa]

        def chunk_op(lk, buf, gsem, ssem, iiX, jjX, prefetch, issue_next):
            pltpu.make_async_copy(h_hbm.at[jjX.at[lk]], buf, gsem).wait()
            prefetch()
            nxt = sb * SLAB + lk + 1
            wt_wait(0)
            mul_half(buf, wb0, 0)
            if issue_next:
                wt_issue(nxt, 0)
            wt_wait(1)
            mul_half(buf, wb1, HC)
            if issue_next:
                wt_issue(nxt, 1)
            pltpu.async_copy(buf, agg.at[iiX.at[lk]], ssem, add=True)

        def pair(k2, carry):
            a = 2 * k2

            def pref_a():
                pltpu.make_async_copy(bB, agg.at[iiP.at[a]], sB).wait()
                pltpu.async_copy(h_hbm.at[jjP.at[a + 1]], bB, gB)

            chunk_op(a, bA, gA, sA, iiP, jjP, pref_a, True)

            def pref_b():
                pltpu.make_async_copy(bA, agg.at[iiP.at[a + 1]], sA).wait()
                pltpu.async_copy(h_hbm.at[jjP.at[a + 2]], bA, gA)

            chunk_op(a + 1, bB, gB, sB, iiP, jjP, pref_b, True)
            return carry

        lax.fori_loop(0, PAIRS, pair, None)

        lk_tail = SLAB - 1
        if sb < NSLAB - 1:
            def pref_tail():
                pltpu.sync_copy(ii_hbm.at[wid].at[pl.ds((sb + 1) * SLAB, SLAB)], iiQ)
                pltpu.sync_copy(jj_hbm.at[wid].at[pl.ds((sb + 1) * SLAB, SLAB)], jjQ)
                pltpu.make_async_copy(bB, agg.at[iiP.at[lk_tail]], sB).wait()
                pltpu.async_copy(h_hbm.at[jjQ.at[0]], bB, gB)
        else:
            def pref_tail():
                pass

        chunk_op(lk_tail, bA, gA, sA, iiP, jjP, pref_tail,
                 sb < NSLAB - 1)

    pltpu.make_async_copy(hbuf0, agg.at[ii0.at[0]], s0).wait()
    pltpu.make_async_copy(hbuf1, agg.at[ii0.at[0]], s1).wait()
    plsc.subcore_barrier()

    osl = pl.ds(sid * NPL, NPL)
    pltpu.sync_copy(agg.at[osl], out_hbm.at[cid].at[osl])


def _tc_update_body(p_ref, w1_ref, b1_ref, w2_ref, b2_ref, x_ref, wn_ref,
                    xo_ref, ho_ref):
    aggb = p_ref[0] + p_ref[1]
    g = jnp.dot(aggb, w1_ref[...], preferred_element_type=jnp.float32) + b1_ref[...]
    v = jnp.dot(jax.nn.softplus(g) - LN2, w2_ref[...],
                preferred_element_type=jnp.float32) + b2_ref[...]
    xn = x_ref[...] + v
    xo_ref[...] = xn
    ho_ref[...] = jnp.dot(xn, wn_ref[...], preferred_element_type=jnp.float32)


_tc_update = pl.pallas_call(
    _tc_update_body,
    grid=(NP // BN,),
    in_specs=[
        pl.BlockSpec((NC, BN, D), lambda i: (0, i, 0)),
        pl.BlockSpec((D, D), lambda i: (0, 0)),
        pl.BlockSpec((1, D), lambda i: (0, 0)),
        pl.BlockSpec((D, D), lambda i: (0, 0)),
        pl.BlockSpec((1, D), lambda i: (0, 0)),
        pl.BlockSpec((BN, D), lambda i: (i, 0)),
        pl.BlockSpec((D, D), lambda i: (0, 0)),
    ],
    out_specs=[
        pl.BlockSpec((BN, D), lambda i: (i, 0)),
        pl.BlockSpec((BN, D), lambda i: (i, 0)),
    ],
    out_shape=[
        jax.ShapeDtypeStruct((NP, D), jnp.float32),
        jax.ShapeDtypeStruct((NP, D), jnp.float32),
    ],
)


def _tc_matmul_body(x_ref, w_ref, o_ref):
    o_ref[...] = jnp.dot(x_ref[...], w_ref[...],
                         preferred_element_type=jnp.float32)


_tc_matmul = pl.pallas_call(
    _tc_matmul_body,
    grid=(NP // BN,),
    in_specs=[
        pl.BlockSpec((BN, D), lambda i: (i, 0)),
        pl.BlockSpec((D, D), lambda i: (0, 0)),
    ],
    out_specs=pl.BlockSpec((BN, D), lambda i: (i, 0)),
    out_shape=jax.ShapeDtypeStruct((NP, D), jnp.float32),
)


def kernel(atomic_numbers, R, idx_i, idx_j, offsets, emb, in2f_W, f2out_W1,
           f2out_b1, f2out_W2, f2out_b2, filt_W1, filt_b1, filt_W2, filt_b2):
    del offsets
    an = atomic_numbers.astype(jnp.int32)
    an_pad = jnp.concatenate([an, jnp.zeros((NP - N,), jnp.int32)])
    ii = idx_i.astype(jnp.int32)
    jj = idx_j.astype(jnp.int32)

    x0, d2 = _sc_embed_geom(an_pad, emb, R.reshape(N * 3), ii, jj)
    d2c = d2.reshape(E, 1)
    rcc = _tc_rcut(d2.reshape(E // 128, 128)).reshape(E, 1)

    centers = jnp.concatenate(
        [jnp.linspace(0.0, CUTOFF, NRBF),
         jnp.zeros((NRBF_P - NRBF,))]).reshape(1, NRBF_P).astype(jnp.float32)
    width = CUTOFF / (NRBF - 1)
    coeff = jnp.concatenate(
        [jnp.full((NRBF,), -0.5 / width**2),
         jnp.zeros((NRBF_P - NRBF,))]).reshape(1, NRBF_P).astype(jnp.float32)
    w1p = jnp.concatenate(
        [filt_W1, jnp.zeros((NI, NRBF_P - NRBF, D), jnp.float32)], axis=1)


    ii3 = ii.reshape(NW, KC, C)
    jj3 = jj.reshape(NW, KC, C)

    wts = [
        _tc_filter(d2c, rcc, centers, coeff, w1p[t],
                   filt_b1[t].reshape(1, D), filt_W2[t],
                   filt_b2[t].reshape(1, D))
        for t in range(NI)
    ]
    x = x0
    h = _tc_matmul(x, in2f_W[0])
    for t in range(NI):
        wt = wts[t]
        parts = _sc_conv(h, wt, ii3, jj3)
        x, h = _tc_update(parts, f2out_W1[t], f2out_b1[t].reshape(1, D),
                          f2out_W2[t], f2out_b2[t].reshape(1, D),
                          x, in2f_W[(t + 1) % NI])
    return x[:N]


# speedup vs baseline: 1.7625x; 1.0571x over previous
import functools
import math

import jax
import jax.numpy as jnp
import numpy as np
from jax import lax
from jax.experimental import pallas as pl
from jax.experimental.pallas import tpu as pltpu
from jax.experimental.pallas import tpu_sc as plsc

N = 10000
E = 320000
D = 128
NRBF = 20
NI = 3
CUTOFF = 5.0
LN2 = math.log(2.0)

NC = 2
NS = 16
NW = NC * NS
L = 16

NP = 10240
NPW = NP // NW
NPL = N // NS
EPW = E // NW
C = 80
KC = EPW // C
SLAB = 25
NSLAB = KC // SLAB
PAIRS = (SLAB - 1) // 2
GC = NPW // C

NRBF_P = 32
BE = 2560
BN = 640

_sc_mesh = plsc.VectorSubcoreMesh(
    core_axis_name="c", subcore_axis_name="s", num_cores=NC, num_subcores=NS)
_sc_params = pltpu.CompilerParams(
    needs_layout_passes=False, use_tc_tiling_on_sc=False)


@functools.partial(
    pl.kernel,
    out_type=(
        jax.ShapeDtypeStruct((NP, D), jnp.float32),
        jax.ShapeDtypeStruct((E,), jnp.float32),
    ),
    mesh=_sc_mesh,
    compiler_params=_sc_params,
    scratch_types=[
        pltpu.VMEM((NPW,), jnp.int32),
        pltpu.VMEM((NPW, D), jnp.float32),
        pltpu.VMEM((N * 3,), jnp.float32),
        pltpu.VMEM((EPW,), jnp.int32),
        pltpu.VMEM((EPW,), jnp.int32),
        pltpu.VMEM((EPW,), jnp.float32),
        pltpu.SemaphoreType.DMA,
    ],
)
def _sc_embed_geom(an_hbm, emb_hbm, r_hbm, ii_hbm, jj_hbm,
                   x0_hbm, d2_hbm,
                   an_v, xrows_v, r_v, ii_v, jj_v, d2_v, sem):
    cid = lax.axis_index("c")
    sid = lax.axis_index("s")
    wid = sid * NC + cid

    nbase = wid * NPW
    pltpu.sync_copy(an_hbm.at[pl.ds(nbase, NPW)], an_v)
    for g in range(GC):
        pltpu.async_copy(
            emb_hbm.at[an_v.at[pl.ds(g * C, C)]],
            xrows_v.at[pl.ds(g * C, C)], sem).wait()
    pltpu.sync_copy(xrows_v, x0_hbm.at[pl.ds(nbase, NPW)])

    ebase = wid * EPW
    pltpu.sync_copy(r_hbm, r_v)
    pltpu.sync_copy(ii_hbm.at[pl.ds(ebase, EPW)], ii_v)
    pltpu.sync_copy(jj_hbm.at[pl.ds(ebase, EPW)], jj_v)
    def body(e, carry):
        sl = pl.ds(e * L, L)
        iv = ii_v[sl] * 3
        jv = jj_v[sl] * 3
        xi = plsc.load_gather(r_v, [iv])
        yi = plsc.load_gather(r_v, [iv + 1])
        zi = plsc.load_gather(r_v, [iv + 2])
        xj = plsc.load_gather(r_v, [jv])
        yj = plsc.load_gather(r_v, [jv + 1])
        zj = plsc.load_gather(r_v, [jv + 2])
        dx = xj - xi
        dy = yj - yi
        dz = zj - zi
        d2_v[sl] = dx * dx + dy * dy + dz * dz
        return carry

    lax.fori_loop(0, EPW // L, body, None)
    pltpu.sync_copy(d2_v, d2_hbm.at[pl.ds(ebase, EPW)])


def _tc_rcut_body(d2_ref, rc_ref, d_ref):
    d = jnp.sqrt(d2_ref[...] + 1e-12)
    rc = 0.5 * (jnp.cos(d * (math.pi / CUTOFF)) + 1.0)
    rc_ref[...] = rc * (d < CUTOFF).astype(jnp.float32)
    d_ref[...] = d


_tc_rcut = pl.pallas_call(
    _tc_rcut_body,
    out_shape=[
        jax.ShapeDtypeStruct((E // 128, 128), jnp.float32),
        jax.ShapeDtypeStruct((E // 128, 128), jnp.float32),
    ],
)


def _tc_filter_body(d_ref, rc_ref, cen_ref, coe_ref, w1_ref, b1_ref, w2_ref,
                    b2_ref, out_ref):
    d = d_ref[...]
    diff = d - cen_ref[...]
    f = jnp.exp(coe_ref[...] * diff * diff)
    g = jnp.dot(f, w1_ref[...], preferred_element_type=jnp.float32) + b1_ref[...]
    a = jax.nn.softplus(g) - LN2
    w = jnp.dot(a, w2_ref[...], preferred_element_type=jnp.float32) + b2_ref[...]
    out_ref[...] = w * rc_ref[...]


_tc_filter = pl.pallas_call(
    _tc_filter_body,
    grid=(E // BE,),
    in_specs=[
        pl.BlockSpec((BE, 1), lambda i: (i, 0)),
        pl.BlockSpec((BE, 1), lambda i: (i, 0)),
        pl.BlockSpec((1, NRBF_P), lambda i: (0, 0)),
        pl.BlockSpec((1, NRBF_P), lambda i: (0, 0)),
        pl.BlockSpec((NRBF_P, D), lambda i: (0, 0)),
        pl.BlockSpec((1, D), lambda i: (0, 0)),
        pl.BlockSpec((D, D), lambda i: (0, 0)),
        pl.BlockSpec((1, D), lambda i: (0, 0)),
    ],
    out_specs=pl.BlockSpec((BE, D), lambda i: (i, 0)),
    out_shape=jax.ShapeDtypeStruct((E, D), jnp.float32),
)


@functools.partial(
    pl.kernel,
    out_type=jax.ShapeDtypeStruct((NC, NP, D), jnp.float32),
    mesh=_sc_mesh,
    compiler_params=_sc_params,
    scratch_types=[
        pltpu.VMEM((SLAB, C), jnp.int32),
        pltpu.VMEM((SLAB, C), jnp.int32),
        pltpu.VMEM((SLAB, C), jnp.int32),
        pltpu.VMEM((SLAB, C), jnp.int32),
        pltpu.VMEM((C, D), jnp.float32),
        pltpu.VMEM((C, D), jnp.float32),
        pltpu.VMEM((C // 2, D), jnp.float32),
        pltpu.VMEM((C // 2, D), jnp.float32),
        pltpu.VMEM_SHARED((N, D), jnp.float32),
        pltpu.SemaphoreType.DMA,
        pltpu.SemaphoreType.DMA,
        pltpu.SemaphoreType.DMA,
        pltpu.SemaphoreType.DMA,
        pltpu.SemaphoreType.DMA,
        pltpu.SemaphoreType.DMA,
    ],
)
def _sc_conv(h_hbm, wt_hbm, ii_hbm, jj_hbm, out_hbm,
             ii0, jj0, ii1, jj1, hbuf0, hbuf1, wb0, wb1, agg,
             g0, g1, s0, s1, w0, w1):
    cid = lax.axis_index("c")
    sid = lax.axis_index("s")
    wid = sid * NC + cid
    ebase = wid * EPW

    def zrow(r, carry):
        for c8 in range(D // L):
            hbuf0[r, pl.ds(c8 * L, L)] = jnp.zeros((L,), jnp.float32)
            hbuf1[r, pl.ds(c8 * L, L)] = jnp.zeros((L,), jnp.float32)
        return carry

    lax.fori_loop(0, C, zrow, None)
    for z in range(NPL // C):
        pltpu.sync_copy(hbuf0, agg.at[pl.ds(sid * NPL + z * C, C)])
    pltpu.sync_copy(hbuf0.at[pl.ds(0, NPL % C)],
                    agg.at[pl.ds(sid * NPL + (NPL // C) * C, NPL % C)])
    pltpu.sync_copy(ii_hbm.at[wid].at[pl.ds(0, SLAB)], ii0)
    pltpu.sync_copy(jj_hbm.at[wid].at[pl.ds(0, SLAB)], jj0)
    plsc.subcore_barrier()

    pltpu.async_copy(hbuf0, agg.at[ii0.at[0]], s0, add=True)
    pltpu.async_copy(hbuf1, agg.at[ii0.at[0]], s1, add=True)
    pltpu.make_async_copy(hbuf0, agg.at[ii0.at[0]], s0).wait()
    pltpu.async_copy(h_hbm.at[jj0.at[0]], hbuf0, g0)

    HC = C // 2
    wbs = (wb0, wb1)
    wsems = (w0, w1)

    def wt_issue(abs_chunk, half):
        off = pl.multiple_of(ebase + abs_chunk * C + half * HC, 8)
        pltpu.async_copy(wt_hbm.at[pl.ds(off, HC)], wbs[half], wsems[half])

    def wt_wait(half):
        off0 = pl.multiple_of(ebase, 8)
        pltpu.make_async_copy(
            wt_hbm.at[pl.ds(off0, HC)], wbs[half], wsems[half]).wait()

    wt_issue(0, 0)
    wt_issue(0, 1)

    bufs = (hbuf0, hbuf1)
    gsems = (g0, g1)
    ssems = (s0, s1)
    slabs = ((ii0, jj0), (ii1, jj1))

    def mul_half(buf, wb, base):
        def mrow(r2, c2):
            for dr in range(2):
                r = r2 * 2 + dr
                for c8 in range(D // L):
                    sl = pl.ds(c8 * L, L)
                    buf[base + r, sl] = buf[base + r, sl] * wb[r, sl]
            return c2

        lax.fori_loop(0, HC // 2, mrow, None)

    for sb in range(NSLAB):
        sa = sb % 2
        iiP, jjP = slabs[sa]
        iiQ, jjQ = slabs[1 - sa]
        bA, bB = bufs[sa], bufs[1 - sa]
        gA, gB = gsems[sa], gsems[1 - sa]
        sA, sB = ssems[sa], ssems[1 - sa]

        def chunk_op(lk, buf, gsem, ssem, iiX, jjX, prefetch, issue_next):
            pltpu.make_async_copy(h_hbm.at[jjX.at[lk]], buf, gsem).wait()
            prefetch()
            nxt = sb * SLAB + lk + 1
            wt_wait(0)
            mul_half(buf, wb0, 0)
            if issue_next:
                wt_issue(nxt, 0)
            wt_wait(1)
            mul_half(buf, wb1, HC)
            if issue_next:
                wt_issue(nxt, 1)
            pltpu.async_copy(buf, agg.at[iiX.at[lk]], ssem, add=True)

        def pair(k2, carry):
            a = 2 * k2

            def pref_a():
                pltpu.make_async_copy(bB, agg.at[iiP.at[a]], sB).wait()
                pltpu.async_copy(h_hbm.at[jjP.at[a + 1]], bB, gB)

            chunk_op(a, bA, gA, sA, iiP, jjP, pref_a, True)

            def pref_b():
                pltpu.make_async_copy(bA, agg.at[iiP.at[a + 1]], sA).wait()
                pltpu.async_copy(h_hbm.at[jjP.at[a + 2]], bA, gA)

            chunk_op(a + 1, bB, gB, sB, iiP, jjP, pref_b, True)
            return carry

        lax.fori_loop(0, PAIRS, pair, None)

        lk_tail = SLAB - 1
        if sb < NSLAB - 1:
            def pref_tail():
                pltpu.sync_copy(ii_hbm.at[wid].at[pl.ds((sb + 1) * SLAB, SLAB)], iiQ)
                pltpu.sync_copy(jj_hbm.at[wid].at[pl.ds((sb + 1) * SLAB, SLAB)], jjQ)
                pltpu.make_async_copy(bB, agg.at[iiP.at[lk_tail]], sB).wait()
                pltpu.async_copy(h_hbm.at[jjQ.at[0]], bB, gB)
        else:
            def pref_tail():
                pass

        chunk_op(lk_tail, bA, gA, sA, iiP, jjP, pref_tail,
                 sb < NSLAB - 1)

    pltpu.make_async_copy(hbuf0, agg.at[ii0.at[0]], s0).wait()
    pltpu.make_async_copy(hbuf1, agg.at[ii0.at[0]], s1).wait()
    plsc.subcore_barrier()

    osl = pl.ds(sid * NPL, NPL)
    pltpu.sync_copy(agg.at[osl], out_hbm.at[cid].at[osl])


def _tc_update_body(p_ref, w1_ref, b1_ref, w2_ref, b2_ref, x_ref, wn_ref,
                    xo_ref, ho_ref):
    aggb = p_ref[0] + p_ref[1]
    g = jnp.dot(aggb, w1_ref[...], preferred_element_type=jnp.float32) + b1_ref[...]
    v = jnp.dot(jax.nn.softplus(g) - LN2, w2_ref[...],
                preferred_element_type=jnp.float32) + b2_ref[...]
    xn = x_ref[...] + v
    xo_ref[...] = xn
    ho_ref[...] = jnp.dot(xn, wn_ref[...], preferred_element_type=jnp.float32)


_tc_update = pl.pallas_call(
    _tc_update_body,
    grid=(NP // BN,),
    in_specs=[
        pl.BlockSpec((NC, BN, D), lambda i: (0, i, 0)),
        pl.BlockSpec((D, D), lambda i: (0, 0)),
        pl.BlockSpec((1, D), lambda i: (0, 0)),
        pl.BlockSpec((D, D), lambda i: (0, 0)),
        pl.BlockSpec((1, D), lambda i: (0, 0)),
        pl.BlockSpec((BN, D), lambda i: (i, 0)),
        pl.BlockSpec((D, D), lambda i: (0, 0)),
    ],
    out_specs=[
        pl.BlockSpec((BN, D), lambda i: (i, 0)),
        pl.BlockSpec((BN, D), lambda i: (i, 0)),
    ],
    out_shape=[
        jax.ShapeDtypeStruct((NP, D), jnp.float32),
        jax.ShapeDtypeStruct((NP, D), jnp.float32),
    ],
)


def _tc_matmul_body(x_ref, w_ref, o_ref):
    o_ref[...] = jnp.dot(x_ref[...], w_ref[...],
                         preferred_element_type=jnp.float32)


_tc_matmul = pl.pallas_call(
    _tc_matmul_body,
    grid=(NP // BN,),
    in_specs=[
        pl.BlockSpec((BN, D), lambda i: (i, 0)),
        pl.BlockSpec((D, D), lambda i: (0, 0)),
    ],
    out_specs=pl.BlockSpec((BN, D), lambda i: (i, 0)),
    out_shape=jax.ShapeDtypeStruct((NP, D), jnp.float32),
)


def kernel(atomic_numbers, R, idx_i, idx_j, offsets, emb, in2f_W, f2out_W1,
           f2out_b1, f2out_W2, f2out_b2, filt_W1, filt_b1, filt_W2, filt_b2):
    del offsets
    an = atomic_numbers.astype(jnp.int32)
    an_pad = jnp.concatenate([an, jnp.zeros((NP - N,), jnp.int32)])
    ii = idx_i.astype(jnp.int32)
    jj = idx_j.astype(jnp.int32)

    x0, d2 = _sc_embed_geom(an_pad, emb, R.reshape(N * 3), ii, jj)
    rcr, dr = _tc_rcut(d2.reshape(E // 128, 128))
    rcc = rcr.reshape(E, 1)
    dc = dr.reshape(E, 1)

    centers = jnp.concatenate(
        [jnp.linspace(0.0, CUTOFF, NRBF),
         jnp.zeros((NRBF_P - NRBF,))]).reshape(1, NRBF_P).astype(jnp.float32)
    width = CUTOFF / (NRBF - 1)
    coeff = jnp.concatenate(
        [jnp.full((NRBF,), -0.5 / width**2),
         jnp.zeros((NRBF_P - NRBF,))]).reshape(1, NRBF_P).astype(jnp.float32)
    w1p = jnp.concatenate(
        [filt_W1, jnp.zeros((NI, NRBF_P - NRBF, D), jnp.float32)], axis=1)


    ii3 = ii.reshape(NW, KC, C)
    jj3 = jj.reshape(NW, KC, C)

    wts = [
        _tc_filter(dc, rcc, centers, coeff, w1p[t],
                   filt_b1[t].reshape(1, D), filt_W2[t],
                   filt_b2[t].reshape(1, D))
        for t in range(NI)
    ]
    x = x0
    h = _tc_matmul(x, in2f_W[0])
    for t in range(NI):
        wt = wts[t]
        parts = _sc_conv(h, wt, ii3, jj3)
        x, h = _tc_update(parts, f2out_W1[t], f2out_b1[t].reshape(1, D),
                          f2out_W2[t], f2out_b2[t].reshape(1, D),
                          x, in2f_W[(t + 1) % NI])
    return x[:N]
